# baseline (speedup 1.0000x reference)
import functools

import jax
import jax.numpy as jnp
from jax import lax
from jax.experimental import pallas as pl
from jax.experimental.pallas import tpu as pltpu
from jax.experimental.pallas import tpu_sc as plsc

ENTITY_SIZE = 32
NUM_BINS = 16
NC, NS, LANES = 2, 16, 16
NW = NC * NS

_SIZES = dict(move=1024, item=512, ability=384, minor=128, se=256, mt=640,
              b01=169, b23=169, b45=169, b6s=78, dmg=2047)
_OFF = {}
_acc = 0
for _k, _v in _SIZES.items():
    _OFF[_k] = _acc
    _acc += _v
ROWS = _acc


def _build_packed_table(table_move, table_item, table_ability, table_status,
                        table_edge_type, table_major, table_minor, table_turn,
                        W_boosts, b_boosts, W_damage, b_damage, W_side, b_side):
    d = ENTITY_SIZE
    vals = jnp.arange(13, dtype=jnp.float32) - 6.0
    boost = vals[None, :, None] * W_boosts[:, None, :]

    def pair(a, b):
        return (a[:, None, :] + b[None, :, :]).reshape(-1, d)

    v = jnp.arange(-1023, 1024, dtype=jnp.int32)
    raw = v / 1023.0
    divisor = 2048.0 / NUM_BINS
    tok = jnp.floor((v + 1023) / divisor)
    tok = jnp.where(v == 0, NUM_BINS + 1, tok)
    onehot = jax.nn.one_hot(tok, NUM_BINS + 1)
    feats = jnp.concatenate([raw[:, None], jnp.abs(raw)[:, None],
                             jnp.sign(v).astype(jnp.float32)[:, None], onehot],
                            axis=-1)
    dmg_rows = feats @ W_damage + (b_damage + b_boosts)[None, :]
    bits = ((jnp.arange(3, dtype=jnp.int32)[:, None]
             & jnp.asarray([1, 2], jnp.int32)[None, :]) != 0).astype(jnp.float32)
    side_rows = jnp.concatenate([jnp.zeros((3, d), jnp.float32),
                                 bits @ W_side + b_side[None, :]], axis=0)
    tab = jnp.concatenate([
        table_move, table_item, table_ability, table_minor,
        pair(table_status, table_edge_type),
        pair(table_major, table_turn),
        pair(boost[0], boost[1]),
        pair(boost[2], boost[3]),
        pair(boost[4], boost[5]),
        pair(boost[6], side_rows),
        dmg_rows,
    ], axis=0)
    tab_bf = tab.astype(jnp.bfloat16).reshape(ROWS, 2, d // 2)
    tab_bf = jnp.swapaxes(tab_bf, 1, 2)
    return lax.bitcast_convert_type(tab_bf, jnp.int32)


def _make_gather_sum(n_tokens):
    per_w = n_tokens // NW
    chunk = 256
    n_chunks = per_w // chunk
    groups = chunk // LANES
    mesh = plsc.VectorSubcoreMesh(core_axis_name="c", subcore_axis_name="s")

    @functools.partial(
        pl.kernel,
        out_type=jax.ShapeDtypeStruct((n_tokens, 16), jnp.int32),
        mesh=mesh,
        scratch_types=[
            pltpu.VMEM((ROWS * 16 // 128, 128), jnp.int32),
            pltpu.VMEM((48, 128), jnp.int32),
            pltpu.VMEM((chunk, 16), jnp.int32),
        ],
        compiler_params=pltpu.CompilerParams(needs_layout_passes=False,
                                             disable_bounds_checks=True),
    )
    def gather_sum(table_hbm, edges_hbm, out_hbm, table_v, edges_v, out_v):
        wid = lax.axis_index("s") * NC + lax.axis_index("c")
        pltpu.sync_copy(table_hbm, table_v)
        lanes = lax.iota(jnp.int32, LANES)

        erows = n_tokens * 19 // 128

        def chunk_body(c, carry):
            base = wid * per_w + c * chunk
            word0 = base * 19
            start_row = jnp.minimum((word0 >> 7) & -8, erows - 48)
            start_row = pl.multiple_of(start_row, 8)
            pltpu.sync_copy(edges_hbm.at[pl.ds(start_row, 48)], edges_v)
            off = word0 - (start_row << 7)

            def group_body(g, carry2):
                tok = lanes + g * LANES
                tok19 = tok * 19 + off

                def fld(f):
                    fw = tok19 + f
                    return plsc.load_gather(edges_v, [fw >> 7, fw & 127])

                has1 = (fld(0) >= 0).astype(jnp.int32)
                rows = [
                    fld(2),
                    _OFF["item"] + fld(3),
                    _OFF["ability"] + fld(4),
                    _OFF["minor"] + fld(7),
                    _OFF["se"] + fld(5) * 16 + fld(8),
                    _OFF["mt"] + fld(6) * 20 + fld(17),
                    _OFF["b01"] + (fld(9) + 6) * 13 + fld(10) + 6,
                    _OFF["b23"] + (fld(11) + 6) * 13 + fld(12) + 6,
                    _OFF["b45"] + (fld(13) + 6) * 13 + fld(14) + 6,
                    _OFF["b6s"] + (fld(15) + 6) * 6 + fld(18) + 3 * has1,
                    _OFF["dmg"] + 1023 + fld(16),
                ]
                rows = [(r >> 3, (r & 7) << 4) for r in rows]
                for w in range(16):
                    wv = jnp.full((LANES,), w, jnp.int32)
                    terms = [
                        plsc.bitcast(
                            plsc.load_gather(table_v, [rhi, rlo + w]),
                            jnp.bfloat16)
                        for rhi, rlo in rows
                    ]
                    while len(terms) > 1:
                        terms = ([terms[i] + terms[i + 1]
                                  for i in range(0, len(terms) - 1, 2)]
                                 + ([terms[-1]] if len(terms) % 2 else []))
                    plsc.store_scatter(out_v, [tok, wv],
                                       plsc.bitcast(terms[0], jnp.int32))
                return carry2

            lax.fori_loop(0, groups, group_body, 0)
            pltpu.sync_copy(out_v, out_hbm.at[pl.ds(base, chunk)])
            return carry

        lax.fori_loop(0, n_chunks, chunk_body, 0)

    return gather_sum


def _tc_combine(edges, emb1, emb2, gsum_i32, W1, W2, bias):
    b, t, d = emb1.shape
    bb = 8
    grid = b // bb
    nb = bb * t

    def body(e_ref, e1_ref, e2_ref, g_ref, w1_ref, w2_ref, b_ref, out_ref):
        m1 = (e_ref[:, :, 0:1] >= 0).astype(jnp.float32)
        m2 = (e_ref[:, :, 1:2] >= 0).astype(jnp.float32)
        p1 = jnp.dot(e1_ref[...].reshape(nb, d), w1_ref[...],
                     preferred_element_type=jnp.float32)
        p2 = jnp.dot(e2_ref[...].reshape(nb, d), w2_ref[...],
                     preferred_element_type=jnp.float32)
        x = g_ref[...]
        lo = pltpu.bitcast(x << 16, jnp.float32)
        hi = pltpu.bitcast(jnp.bitwise_and(x, jnp.int32(-65536)), jnp.float32)
        g32 = jnp.concatenate([lo, hi], axis=1)
        out_ref[...] = (m1 * p1.reshape(bb, t, d) + m2 * p2.reshape(bb, t, d)
                        + (g32 + b_ref[...]).reshape(bb, t, d))

    return pl.pallas_call(
        body,
        grid=(grid,),
        in_specs=[
            pl.BlockSpec((bb, t, 19), lambda i: (i, 0, 0)),
            pl.BlockSpec((bb, t, d), lambda i: (i, 0, 0)),
            pl.BlockSpec((bb, t, d), lambda i: (i, 0, 0)),
            pl.BlockSpec((nb, 16), lambda i: (i, 0)),
            pl.BlockSpec((d, d), lambda i: (0, 0)),
            pl.BlockSpec((d, d), lambda i: (0, 0)),
            pl.BlockSpec((1, d), lambda i: (0, 0)),
        ],
        out_specs=pl.BlockSpec((bb, t, d), lambda i: (i, 0, 0)),
        out_shape=jax.ShapeDtypeStruct((b, t, d), jnp.float32),
    )(edges, emb1, emb2, gsum_i32, W1, W2, bias)


def kernel(edges, poke1_embeddings, poke2_embeddings, W1, b1, W2, b2,
           table_move, table_item, table_ability, table_status,
           table_edge_type, table_major, table_minor, table_turn,
           W_boosts, b_boosts, W_damage, b_damage, W_side, b_side):
    b, t, _ = edges.shape
    d = ENTITY_SIZE
    n = b * t
    packed = _build_packed_table(
        table_move, table_item, table_ability, table_status, table_edge_type,
        table_major, table_minor, table_turn, W_boosts, b_boosts,
        W_damage, b_damage, W_side, b_side)
    gsum_i32 = _make_gather_sum(n)(packed.reshape(ROWS * 16 // 128, 128),
                                   edges.reshape(n * 19 // 128, 128))
    return _tc_combine(edges, poke1_embeddings, poke2_embeddings,
                       gsum_i32, W1, W2, (b1 + b2)[None, :])


# speedup vs baseline: 1.3781x; 1.3781x over previous
import functools

import jax
import jax.numpy as jnp
from jax import lax
from jax.experimental import pallas as pl
from jax.experimental.pallas import tpu as pltpu
from jax.experimental.pallas import tpu_sc as plsc

ENTITY_SIZE = 32
NUM_BINS = 16
NC, NS, LANES = 2, 16, 16
NW = NC * NS

_SIZES = dict(move=1024, item=512, ability=384, minor=128, se=256, mt=640,
              b01=169, b23=169, b45=169, b6s=78, dmg=2047)
_OFF = {}
_acc = 0
for _k, _v in _SIZES.items():
    _OFF[_k] = _acc
    _acc += _v
ROWS = _acc


def _build_packed_table(table_move, table_item, table_ability, table_status,
                        table_edge_type, table_major, table_minor, table_turn,
                        W_boosts, b_boosts, W_damage, b_damage, W_side, b_side):
    d = ENTITY_SIZE
    vals = jnp.arange(13, dtype=jnp.float32) - 6.0
    boost = vals[None, :, None] * W_boosts[:, None, :]

    def pair(a, b):
        return (a[:, None, :] + b[None, :, :]).reshape(-1, d)

    v = jnp.arange(-1023, 1024, dtype=jnp.int32)
    raw = v / 1023.0
    divisor = 2048.0 / NUM_BINS
    tok = jnp.floor((v + 1023) / divisor)
    tok = jnp.where(v == 0, NUM_BINS + 1, tok)
    onehot = jax.nn.one_hot(tok, NUM_BINS + 1)
    feats = jnp.concatenate([raw[:, None], jnp.abs(raw)[:, None],
                             jnp.sign(v).astype(jnp.float32)[:, None], onehot],
                            axis=-1)
    dmg_rows = feats @ W_damage + (b_damage + b_boosts)[None, :]
    bits = ((jnp.arange(3, dtype=jnp.int32)[:, None]
             & jnp.asarray([1, 2], jnp.int32)[None, :]) != 0).astype(jnp.float32)
    side_rows = jnp.concatenate([jnp.zeros((3, d), jnp.float32),
                                 bits @ W_side + b_side[None, :]], axis=0)
    tab = jnp.concatenate([
        table_move, table_item, table_ability, table_minor,
        pair(table_status, table_edge_type),
        pair(table_major, table_turn),
        pair(boost[0], boost[1]),
        pair(boost[2], boost[3]),
        pair(boost[4], boost[5]),
        pair(boost[6], side_rows),
        dmg_rows,
    ], axis=0)
    tab_bf = tab.astype(jnp.bfloat16).reshape(ROWS, 2, d // 2)
    tab_bf = jnp.swapaxes(tab_bf, 1, 2)
    packed = lax.bitcast_convert_type(tab_bf, jnp.int32)
    r = jnp.arange(ROWS, dtype=jnp.int32)[:, None]
    c = jnp.arange(16, dtype=jnp.int32)[None, :]
    return jnp.take_along_axis(packed, c ^ (r & 15), axis=1)


def _make_gather_sum(n_tokens):
    per_w = n_tokens // NW
    chunk = 256
    n_chunks = per_w // chunk
    groups = chunk // LANES
    mesh = plsc.VectorSubcoreMesh(core_axis_name="c", subcore_axis_name="s")

    @functools.partial(
        pl.kernel,
        out_type=jax.ShapeDtypeStruct((n_tokens, 16), jnp.int32),
        mesh=mesh,
        scratch_types=[
            pltpu.VMEM((ROWS * 16 // 128, 128), jnp.int32),
            pltpu.VMEM((48, 128), jnp.int32),
            pltpu.VMEM((chunk, 16), jnp.int32),
        ],
        compiler_params=pltpu.CompilerParams(needs_layout_passes=False,
                                             disable_bounds_checks=True),
    )
    def gather_sum(table_hbm, edges_hbm, out_hbm, table_v, edges_v, out_v):
        wid = lax.axis_index("s") * NC + lax.axis_index("c")
        pltpu.sync_copy(table_hbm, table_v)
        lanes = lax.iota(jnp.int32, LANES)

        erows = n_tokens * 19 // 128

        def chunk_body(c, carry):
            base = wid * per_w + c * chunk
            word0 = base * 19
            start_row = jnp.minimum((word0 >> 7) & -8, erows - 48)
            start_row = pl.multiple_of(start_row, 8)
            pltpu.sync_copy(edges_hbm.at[pl.ds(start_row, 48)], edges_v)
            off = word0 - (start_row << 7)

            def group_body(g, carry2):
                tok = lanes + g * LANES
                tok19 = tok * 19 + off

                def fld(f):
                    fw = tok19 + f
                    return plsc.load_gather(edges_v, [fw >> 7, fw & 127])

                has1 = (fld(0) >= 0).astype(jnp.int32)
                rows = [
                    fld(2),
                    _OFF["item"] + fld(3),
                    _OFF["ability"] + fld(4),
                    _OFF["minor"] + fld(7),
                    _OFF["se"] + fld(5) * 16 + fld(8),
                    _OFF["mt"] + fld(6) * 20 + fld(17),
                    _OFF["b01"] + (fld(9) + 6) * 13 + fld(10) + 6,
                    _OFF["b23"] + (fld(11) + 6) * 13 + fld(12) + 6,
                    _OFF["b45"] + (fld(13) + 6) * 13 + fld(14) + 6,
                    _OFF["b6s"] + (fld(15) + 6) * 6 + fld(18) + 3 * has1,
                    _OFF["dmg"] + 1023 + fld(16),
                ]
                rows = [(r >> 3, ((r & 7) << 4) | (r & 15)) for r in rows]
                for w in range(16):
                    wv = jnp.full((LANES,), w, jnp.int32)
                    terms = [
                        plsc.bitcast(
                            plsc.load_gather(table_v, [rhi, cb ^ w]),
                            jnp.bfloat16)
                        for rhi, cb in rows
                    ]
                    while len(terms) > 1:
                        terms = ([terms[i] + terms[i + 1]
                                  for i in range(0, len(terms) - 1, 2)]
                                 + ([terms[-1]] if len(terms) % 2 else []))
                    plsc.store_scatter(out_v, [tok, wv],
                                       plsc.bitcast(terms[0], jnp.int32))
                return carry2

            lax.fori_loop(0, groups, group_body, 0)
            pltpu.sync_copy(out_v, out_hbm.at[pl.ds(base, chunk)])
            return carry

        lax.fori_loop(0, n_chunks, chunk_body, 0)

    return gather_sum


def _tc_combine(edges, emb1, emb2, gsum_i32, W1, W2, bias):
    b, t, d = emb1.shape
    bb = 8
    grid = b // bb
    nb = bb * t

    def body(e_ref, e1_ref, e2_ref, g_ref, w1_ref, w2_ref, b_ref, out_ref):
        m1 = (e_ref[:, :, 0:1] >= 0).astype(jnp.float32)
        m2 = (e_ref[:, :, 1:2] >= 0).astype(jnp.float32)
        p1 = jnp.dot(e1_ref[...].reshape(nb, d), w1_ref[...],
                     preferred_element_type=jnp.float32)
        p2 = jnp.dot(e2_ref[...].reshape(nb, d), w2_ref[...],
                     preferred_element_type=jnp.float32)
        x = g_ref[...]
        lo = pltpu.bitcast(x << 16, jnp.float32)
        hi = pltpu.bitcast(jnp.bitwise_and(x, jnp.int32(-65536)), jnp.float32)
        g32 = jnp.concatenate([lo, hi], axis=1)
        out_ref[...] = (m1 * p1.reshape(bb, t, d) + m2 * p2.reshape(bb, t, d)
                        + (g32 + b_ref[...]).reshape(bb, t, d))

    return pl.pallas_call(
        body,
        grid=(grid,),
        in_specs=[
            pl.BlockSpec((bb, t, 19), lambda i: (i, 0, 0)),
            pl.BlockSpec((bb, t, d), lambda i: (i, 0, 0)),
            pl.BlockSpec((bb, t, d), lambda i: (i, 0, 0)),
            pl.BlockSpec((nb, 16), lambda i: (i, 0)),
            pl.BlockSpec((d, d), lambda i: (0, 0)),
            pl.BlockSpec((d, d), lambda i: (0, 0)),
            pl.BlockSpec((1, d), lambda i: (0, 0)),
        ],
        out_specs=pl.BlockSpec((bb, t, d), lambda i: (i, 0, 0)),
        out_shape=jax.ShapeDtypeStruct((b, t, d), jnp.float32),
    )(edges, emb1, emb2, gsum_i32, W1, W2, bias)


def kernel(edges, poke1_embeddings, poke2_embeddings, W1, b1, W2, b2,
           table_move, table_item, table_ability, table_status,
           table_edge_type, table_major, table_minor, table_turn,
           W_boosts, b_boosts, W_damage, b_damage, W_side, b_side):
    b, t, _ = edges.shape
    d = ENTITY_SIZE
    n = b * t
    packed = _build_packed_table(
        table_move, table_item, table_ability, table_status, table_edge_type,
        table_major, table_minor, table_turn, W_boosts, b_boosts,
        W_damage, b_damage, W_side, b_side)
    gsum_i32 = _make_gather_sum(n)(packed.reshape(ROWS * 16 // 128, 128),
                                   edges.reshape(n * 19 // 128, 128))
    return _tc_combine(edges, poke1_embeddings, poke2_embeddings,
                       gsum_i32, W1, W2, (b1 + b2)[None, :])


# speedup vs baseline: 1.4308x; 1.0382x over previous
import functools

import jax
import jax.numpy as jnp
from jax import lax
from jax.experimental import pallas as pl
from jax.experimental.pallas import tpu as pltpu
from jax.experimental.pallas import tpu_sc as plsc

ENTITY_SIZE = 32
NUM_BINS = 16
NC, NS, LANES = 2, 16, 16
NW = NC * NS

_SIZES = dict(move=1024, item=512, ability=384, minor=128, se=256, mt=640,
              b01=169, b23=169, b45=169, b6s=78, dmg=2047)
_OFF = {}
_acc = 0
for _k, _v in _SIZES.items():
    _OFF[_k] = _acc
    _acc += _v
ROWS = _acc


def _build_packed_table(table_move, table_item, table_ability, table_status,
                        table_edge_type, table_major, table_minor, table_turn,
                        W_boosts, b_boosts, W_damage, b_damage, W_side, b_side):
    d = ENTITY_SIZE
    vals = jnp.arange(13, dtype=jnp.float32) - 6.0
    boost = vals[None, :, None] * W_boosts[:, None, :]

    def pair(a, b):
        return (a[:, None, :] + b[None, :, :]).reshape(-1, d)

    v = jnp.arange(-1023, 1024, dtype=jnp.int32)
    raw = v / 1023.0
    divisor = 2048.0 / NUM_BINS
    tok = jnp.floor((v + 1023) / divisor)
    tok = jnp.where(v == 0, NUM_BINS + 1, tok)
    onehot = jax.nn.one_hot(tok, NUM_BINS + 1)
    feats = jnp.concatenate([raw[:, None], jnp.abs(raw)[:, None],
                             jnp.sign(v).astype(jnp.float32)[:, None], onehot],
                            axis=-1)
    dmg_rows = feats @ W_damage + (b_damage + b_boosts)[None, :]
    bits = ((jnp.arange(3, dtype=jnp.int32)[:, None]
             & jnp.asarray([1, 2], jnp.int32)[None, :]) != 0).astype(jnp.float32)
    side_rows = jnp.concatenate([jnp.zeros((3, d), jnp.float32),
                                 bits @ W_side + b_side[None, :]], axis=0)
    tab = jnp.concatenate([
        table_move, table_item, table_ability, table_minor,
        pair(table_status, table_edge_type),
        pair(table_major, table_turn),
        pair(boost[0], boost[1]),
        pair(boost[2], boost[3]),
        pair(boost[4], boost[5]),
        pair(boost[6], side_rows),
        dmg_rows,
    ], axis=0)
    tab_bf = tab.astype(jnp.bfloat16).reshape(ROWS, 2, d // 2)
    tab_bf = jnp.swapaxes(tab_bf, 1, 2)
    packed = lax.bitcast_convert_type(tab_bf, jnp.int32)
    r = jnp.arange(ROWS, dtype=jnp.int32)[:, None]
    c = jnp.arange(16, dtype=jnp.int32)[None, :]
    return jnp.take_along_axis(packed, c ^ (r & 15), axis=1)


def _make_gather_sum(n_tokens):
    per_w = n_tokens // NW
    chunk = 256
    n_chunks = per_w // chunk
    groups = chunk // LANES
    mesh = plsc.VectorSubcoreMesh(core_axis_name="c", subcore_axis_name="s")

    @functools.partial(
        pl.kernel,
        out_type=jax.ShapeDtypeStruct((n_tokens, 16), jnp.int32),
        mesh=mesh,
        scratch_types=[
            pltpu.VMEM((ROWS * 16 // 128, 128), jnp.int32),
            pltpu.VMEM((48, 128), jnp.int32),
            pltpu.VMEM((chunk, 16), jnp.int32),
        ],
        compiler_params=pltpu.CompilerParams(needs_layout_passes=False,
                                             disable_bounds_checks=True),
    )
    def gather_sum(table_hbm, edges_hbm, out_hbm, table_v, edges_v, out_v):
        wid = lax.axis_index("s") * NC + lax.axis_index("c")
        pltpu.sync_copy(table_hbm, table_v)
        lanes = lax.iota(jnp.int32, LANES)

        erows = n_tokens * 19 // 128

        def chunk_body(c, carry):
            base = wid * per_w + c * chunk
            word0 = base * 19
            start_row = jnp.minimum((word0 >> 7) & -8, erows - 48)
            start_row = pl.multiple_of(start_row, 8)
            pltpu.sync_copy(edges_hbm.at[pl.ds(start_row, 48)], edges_v)
            off = word0 - (start_row << 7)

            def group_body(g, carry2):
                tok = lanes + g * LANES
                tok19 = tok * 19 + off

                def fld(f):
                    fw = tok19 + f
                    return plsc.load_gather(edges_v, [fw >> 7, fw & 127])

                has1 = (fld(0) >= 0).astype(jnp.int32)
                rows = [
                    fld(2),
                    _OFF["item"] + fld(3),
                    _OFF["ability"] + fld(4),
                    _OFF["minor"] + fld(7),
                    _OFF["se"] + fld(5) * 16 + fld(8),
                    _OFF["mt"] + fld(6) * 20 + fld(17),
                    _OFF["b01"] + (fld(9) + 6) * 13 + fld(10) + 6,
                    _OFF["b23"] + (fld(11) + 6) * 13 + fld(12) + 6,
                    _OFF["b45"] + (fld(13) + 6) * 13 + fld(14) + 6,
                    _OFF["b6s"] + (fld(15) + 6) * 6 + fld(18) + 3 * has1,
                    _OFF["dmg"] + 1023 + fld(16),
                ]
                rows = [(r >> 3, ((r & 7) << 4) | (r & 15)) for r in rows]
                for w in range(16):
                    wv = jnp.full((LANES,), w, jnp.int32)
                    terms = [
                        plsc.bitcast(
                            plsc.load_gather(table_v, [rhi, cb ^ w]),
                            jnp.bfloat16)
                        for rhi, cb in rows
                    ]
                    while len(terms) > 1:
                        terms = ([terms[i] + terms[i + 1]
                                  for i in range(0, len(terms) - 1, 2)]
                                 + ([terms[-1]] if len(terms) % 2 else []))
                    plsc.store_scatter(out_v, [tok, wv],
                                       plsc.bitcast(terms[0], jnp.int32))
                return carry2

            lax.fori_loop(0, groups, group_body, 0)
            pltpu.sync_copy(out_v, out_hbm.at[pl.ds(base, chunk)])
            return carry

        lax.fori_loop(0, n_chunks, chunk_body, 0)

    return gather_sum


def _tc_combine(masks_t, emb1, emb2, gsum_i32, W1, W2, bias):
    b, t, d = emb1.shape
    bb = 16
    grid = b // bb
    nb = bb * t

    def body(m_ref, e1_ref, e2_ref, g_ref, w1_ref, w2_ref, b_ref, out_ref):
        m = m_ref[...]
        m1 = (m[0] >= 0).astype(jnp.float32)[:, :, None]
        m2 = (m[1] >= 0).astype(jnp.float32)[:, :, None]
        p1 = jnp.dot(e1_ref[...].reshape(nb, d), w1_ref[...],
                     preferred_element_type=jnp.float32)
        p2 = jnp.dot(e2_ref[...].reshape(nb, d), w2_ref[...],
                     preferred_element_type=jnp.float32)
        x = g_ref[...]
        lo = pltpu.bitcast(x << 16, jnp.float32)
        hi = pltpu.bitcast(jnp.bitwise_and(x, jnp.int32(-65536)), jnp.float32)
        g32 = jnp.concatenate([lo, hi], axis=1)
        out_ref[...] = (m1 * p1.reshape(bb, t, d) + m2 * p2.reshape(bb, t, d)
                        + (g32 + b_ref[...]).reshape(bb, t, d))

    return pl.pallas_call(
        body,
        grid=(grid,),
        in_specs=[
            pl.BlockSpec((2, bb, t), lambda i: (0, i, 0)),
            pl.BlockSpec((bb, t, d), lambda i: (i, 0, 0)),
            pl.BlockSpec((bb, t, d), lambda i: (i, 0, 0)),
            pl.BlockSpec((nb, 16), lambda i: (i, 0)),
            pl.BlockSpec((d, d), lambda i: (0, 0)),
            pl.BlockSpec((d, d), lambda i: (0, 0)),
            pl.BlockSpec((1, d), lambda i: (0, 0)),
        ],
        out_specs=pl.BlockSpec((bb, t, d), lambda i: (i, 0, 0)),
        out_shape=jax.ShapeDtypeStruct((b, t, d), jnp.float32),
    )(masks_t, emb1, emb2, gsum_i32, W1, W2, bias)


def kernel(edges, poke1_embeddings, poke2_embeddings, W1, b1, W2, b2,
           table_move, table_item, table_ability, table_status,
           table_edge_type, table_major, table_minor, table_turn,
           W_boosts, b_boosts, W_damage, b_damage, W_side, b_side):
    b, t, _ = edges.shape
    d = ENTITY_SIZE
    n = b * t
    packed = _build_packed_table(
        table_move, table_item, table_ability, table_status, table_edge_type,
        table_major, table_minor, table_turn, W_boosts, b_boosts,
        W_damage, b_damage, W_side, b_side)
    gsum_i32 = _make_gather_sum(n)(packed.reshape(ROWS * 16 // 128, 128),
                                   edges.reshape(n * 19 // 128, 128))
    masks_t = jnp.transpose(edges, (2, 0, 1))[0:2]
    return _tc_combine(masks_t, poke1_embeddings, poke2_embeddings,
                       gsum_i32, W1, W2, (b1 + b2)[None, :])


# speedup vs baseline: 1.6301x; 1.1393x over previous
import functools

import jax
import jax.numpy as jnp
from jax import lax
from jax.experimental import pallas as pl
from jax.experimental.pallas import tpu as pltpu
from jax.experimental.pallas import tpu_sc as plsc

ENTITY_SIZE = 32
NUM_BINS = 16
NC, NS, LANES = 2, 16, 16
NW = NC * NS

_SIZES = dict(move=1024, item=512, ability=384, minor=128, se=256, mt=640,
              b01=169, b23=169, b45=169, b6s=78, dmg=2047)
_OFF = {}
_acc = 0
for _k, _v in _SIZES.items():
    _OFF[_k] = _acc
    _acc += _v
ROWS = _acc


def _build_packed_table(table_move, table_item, table_ability, table_status,
                        table_edge_type, table_major, table_minor, table_turn,
                        W_boosts, b_boosts, W_damage, b_damage, W_side, b_side):
    d = ENTITY_SIZE
    vals = jnp.arange(13, dtype=jnp.float32) - 6.0
    boost = vals[None, :, None] * W_boosts[:, None, :]

    def pair(a, b):
        return (a[:, None, :] + b[None, :, :]).reshape(-1, d)

    v = jnp.arange(-1023, 1024, dtype=jnp.int32)
    raw = v / 1023.0
    divisor = 2048.0 / NUM_BINS
    tok = jnp.floor((v + 1023) / divisor)
    tok = jnp.where(v == 0, NUM_BINS + 1, tok)
    onehot = jax.nn.one_hot(tok, NUM_BINS + 1)
    feats = jnp.concatenate([raw[:, None], jnp.abs(raw)[:, None],
                             jnp.sign(v).astype(jnp.float32)[:, None], onehot],
                            axis=-1)
    dmg_rows = feats @ W_damage + (b_damage + b_boosts)[None, :]
    bits = ((jnp.arange(3, dtype=jnp.int32)[:, None]
             & jnp.asarray([1, 2], jnp.int32)[None, :]) != 0).astype(jnp.float32)
    side_rows = jnp.concatenate([jnp.zeros((3, d), jnp.float32),
                                 bits @ W_side + b_side[None, :]], axis=0)
    tab = jnp.concatenate([
        table_move, table_item, table_ability, table_minor,
        pair(table_status, table_edge_type),
        pair(table_major, table_turn),
        pair(boost[0], boost[1]),
        pair(boost[2], boost[3]),
        pair(boost[4], boost[5]),
        pair(boost[6], side_rows),
        dmg_rows,
    ], axis=0)
    tab_bf = tab.astype(jnp.bfloat16).reshape(ROWS, 2, d // 2)
    tab_bf = jnp.swapaxes(tab_bf, 1, 2)
    packed = lax.bitcast_convert_type(tab_bf, jnp.int32)
    r = jnp.arange(ROWS, dtype=jnp.int32)[:, None]
    c = jnp.arange(16, dtype=jnp.int32)[None, :]
    return jnp.take_along_axis(packed, c ^ (r & 15), axis=1)


def _make_gather_sum(n_tokens):
    per_w = n_tokens // NW
    chunk = 256
    n_chunks = per_w // chunk
    groups = chunk // LANES
    mesh = plsc.VectorSubcoreMesh(core_axis_name="c", subcore_axis_name="s")

    @functools.partial(
        pl.kernel,
        out_type=jax.ShapeDtypeStruct((16, n_tokens), jnp.int32),
        mesh=mesh,
        scratch_types=[
            pltpu.VMEM((ROWS * 16 // 128, 128), jnp.int32),
            pltpu.VMEM((48, 128), jnp.int32),
            pltpu.VMEM((16, chunk), jnp.int32),
        ],
        compiler_params=pltpu.CompilerParams(needs_layout_passes=False,
                                             disable_bounds_checks=True),
    )
    def gather_sum(table_hbm, edges_hbm, out_hbm, table_v, edges_v, out_v):
        wid = lax.axis_index("s") * NC + lax.axis_index("c")
        pltpu.sync_copy(table_hbm, table_v)
        lanes = lax.iota(jnp.int32, LANES)

        erows = n_tokens * 19 // 128

        def chunk_body(c, carry):
            base = wid * per_w + c * chunk
            word0 = base * 19
            start_row = jnp.minimum((word0 >> 7) & -8, erows - 48)
            start_row = pl.multiple_of(start_row, 8)
            pltpu.sync_copy(edges_hbm.at[pl.ds(start_row, 48)], edges_v)
            off = word0 - (start_row << 7)

            def group_body(g, carry2):
                tok = lanes + g * LANES
                tok19 = tok * 19 + off

                def fld(f):
                    fw = tok19 + f
                    return plsc.load_gather(edges_v, [fw >> 7, fw & 127])

                has1 = (fld(0) >= 0).astype(jnp.int32)
                rows = [
                    fld(2),
                    _OFF["item"] + fld(3),
                    _OFF["ability"] + fld(4),
                    _OFF["minor"] + fld(7),
                    _OFF["se"] + fld(5) * 16 + fld(8),
                    _OFF["mt"] + fld(6) * 20 + fld(17),
                    _OFF["b01"] + (fld(9) + 6) * 13 + fld(10) + 6,
                    _OFF["b23"] + (fld(11) + 6) * 13 + fld(12) + 6,
                    _OFF["b45"] + (fld(13) + 6) * 13 + fld(14) + 6,
                    _OFF["b6s"] + (fld(15) + 6) * 6 + fld(18) + 3 * has1,
                    _OFF["dmg"] + 1023 + fld(16),
                ]
                rows = [(r >> 3, ((r & 7) << 4) | (r & 15)) for r in rows]
                for w in range(16):
                    terms = [
                        plsc.bitcast(
                            plsc.load_gather(table_v, [rhi, cb ^ w]),
                            jnp.bfloat16)
                        for rhi, cb in rows
                    ]
                    while len(terms) > 1:
                        terms = ([terms[i] + terms[i + 1]
                                  for i in range(0, len(terms) - 1, 2)]
                                 + ([terms[-1]] if len(terms) % 2 else []))
                    out_v[w, pl.ds(g * LANES, LANES)] = plsc.bitcast(
                        terms[0], jnp.int32)
                return carry2

            lax.fori_loop(0, groups, group_body, 0)
            pltpu.sync_copy(out_v, out_hbm.at[:, pl.ds(base, chunk)])
            return carry

        lax.fori_loop(0, n_chunks, chunk_body, 0)

    return gather_sum


def _tc_combine(masks_t, emb1, emb2, gsum_i32, W1, W2, bias):
    b, t, d = emb1.shape
    bb = 16
    grid = b // bb
    nb = bb * t

    def body(m_ref, e1_ref, e2_ref, g_ref, w1_ref, w2_ref, b_ref, eye_ref,
             out_ref):
        m = m_ref[...]
        m1 = (m[0] >= 0).astype(jnp.float32)[:, :, None]
        m2 = (m[1] >= 0).astype(jnp.float32)[:, :, None]
        p1 = jnp.dot(e1_ref[...].reshape(nb, d), w1_ref[...],
                     preferred_element_type=jnp.float32)
        p2 = jnp.dot(e2_ref[...].reshape(nb, d), w2_ref[...],
                     preferred_element_type=jnp.float32)
        x = g_ref[...]
        lo = pltpu.bitcast(x << 16, jnp.float32)
        hi = pltpu.bitcast(jnp.bitwise_and(x, jnp.int32(-65536)), jnp.float32)
        g32t = jnp.concatenate([lo, hi], axis=0)
        g32 = lax.dot_general(g32t, eye_ref[...], (((0,), (0,)), ((), ())),
                              preferred_element_type=jnp.float32)
        out_ref[...] = (m1 * p1.reshape(bb, t, d) + m2 * p2.reshape(bb, t, d)
                        + (g32 + b_ref[...]).reshape(bb, t, d))

    return pl.pallas_call(
        body,
        grid=(grid,),
        in_specs=[
            pl.BlockSpec((2, bb, t), lambda i: (0, i, 0)),
            pl.BlockSpec((bb, t, d), lambda i: (i, 0, 0)),
            pl.BlockSpec((bb, t, d), lambda i: (i, 0, 0)),
            pl.BlockSpec((16, nb), lambda i: (0, i)),
            pl.BlockSpec((d, d), lambda i: (0, 0)),
            pl.BlockSpec((d, d), lambda i: (0, 0)),
            pl.BlockSpec((1, d), lambda i: (0, 0)),
            pl.BlockSpec((d, d), lambda i: (0, 0)),
        ],
        out_specs=pl.BlockSpec((bb, t, d), lambda i: (i, 0, 0)),
        out_shape=jax.ShapeDtypeStruct((b, t, d), jnp.float32),
    )(masks_t, emb1, emb2, gsum_i32, W1, W2, bias,
      jnp.eye(d, dtype=jnp.float32))


def kernel(edges, poke1_embeddings, poke2_embeddings, W1, b1, W2, b2,
           table_move, table_item, table_ability, table_status,
           table_edge_type, table_major, table_minor, table_turn,
           W_boosts, b_boosts, W_damage, b_damage, W_side, b_side):
    b, t, _ = edges.shape
    d = ENTITY_SIZE
    n = b * t
    packed = _build_packed_table(
        table_move, table_item, table_ability, table_status, table_edge_type,
        table_major, table_minor, table_turn, W_boosts, b_boosts,
        W_damage, b_damage, W_side, b_side)
    gsum_i32 = _make_gather_sum(n)(packed.reshape(ROWS * 16 // 128, 128),
                                   edges.reshape(n * 19 // 128, 128))
    masks_t = jnp.transpose(edges, (2, 0, 1))[0:2]
    return _tc_combine(masks_t, poke1_embeddings, poke2_embeddings,
                       gsum_i32, W1, W2, (b1 + b2)[None, :])
